# baseline (speedup 1.0000x reference)
import functools

import jax
import jax.numpy as jnp
from jax import lax
from jax.experimental import pallas as pl
from jax.experimental.pallas import tpu as pltpu
from jax.experimental.pallas import tpu_sc as plsc

NUM_TOKENS = 1000000
DIM = 64
BATCH = 4096
SEQ = 200

NC = 2
NS = 16
NW = NC * NS

TOTAL = BATCH * SEQ
IPG = 128
NCHUNK = TOTAL // IPG
CPW = NCHUNK // NW
BBLK = BATCH // IPG

_mesh = plsc.VectorSubcoreMesh(core_axis_name="c", subcore_axis_name="s")


@functools.partial(
    pl.kernel,
    out_type=jax.ShapeDtypeStruct((SEQ, DIM // 8, BBLK, 8, IPG), jnp.float32),
    mesh=_mesh,
    scratch_types=[
        pltpu.VMEM((CPW, IPG), jnp.int32),
        pltpu.VMEM((IPG, DIM), jnp.float32),
        pltpu.VMEM((IPG, DIM), jnp.float32),
        pltpu.VMEM((DIM // 8, 8, IPG), jnp.float32),
        pltpu.VMEM((DIM // 8, 8, IPG), jnp.float32),
        pltpu.SemaphoreType.DMA,
        pltpu.SemaphoreType.DMA,
        pltpu.SemaphoreType.DMA,
        pltpu.SemaphoreType.DMA,
    ],
    compiler_params=pltpu.CompilerParams(
        use_tc_tiling_on_sc=False, needs_layout_passes=False
    ),
)
def _emb_lookup(table_hbm, idx_hbm, out_hbm, idx_v, buf0, buf1, bt0, bt1,
                gsem0, gsem1, osem0, osem1):
    wid = lax.axis_index("s") * NC + lax.axis_index("c")
    c_base = wid * CPW
    pltpu.sync_copy(idx_hbm.at[pl.ds(c_base, CPW)], idx_v)

    iota = lax.iota(jnp.int32, 16)
    bg_idx = [iota + 16 * bg for bg in range(8)]

    def fire_gather(g, buf, gsem):
        pltpu.async_copy(table_hbm.at[idx_v.at[g]], buf, gsem)

    def wait_gather(buf, gsem):
        pltpu.make_async_copy(table_hbm.at[idx_v.at[0]], buf, gsem).wait()

    def transpose(buf, bt):
        def dloop(d, carry):
            dcol = jnp.full((16,), d, jnp.int32)
            dt = d // 8
            d8 = lax.rem(d, 8)
            for bg in range(8):
                bt[dt, d8, pl.ds(16 * bg, 16)] = plsc.load_gather(
                    buf, [bg_idx[bg], dcol]
                )
            return carry

        lax.fori_loop(0, DIM, dloop, 0)

    def fire_out(g, bt, osem):
        c = c_base + g
        s = c // BBLK
        bc = lax.rem(c, BBLK)
        pltpu.async_copy(bt, out_hbm.at[s, :, bc], osem)

    def wait_out(bt, osem):
        pltpu.make_async_copy(bt, out_hbm.at[0, :, 0], osem).wait()

    fire_gather(0, buf0, gsem0)

    def pair_body(t, carry):
        g0 = 2 * t
        fire_gather(g0 + 1, buf1, gsem1)
        wait_gather(buf0, gsem0)

        @pl.when(t > 0)
        def _():
            wait_out(bt0, osem0)

        transpose(buf0, bt0)
        fire_out(g0, bt0, osem0)

        @pl.when(g0 + 2 < CPW)
        def _():
            fire_gather(g0 + 2, buf0, gsem0)

        wait_gather(buf1, gsem1)

        @pl.when(t > 0)
        def _():
            wait_out(bt1, osem1)

        transpose(buf1, bt1)
        fire_out(g0 + 1, bt1, osem1)
        return carry

    lax.fori_loop(0, CPW // 2, pair_body, 0)
    wait_out(bt0, osem0)
    wait_out(bt1, osem1)


def kernel(x, emb_weight):
    idx = x.T.reshape(NCHUNK, IPG)
    out5 = _emb_lookup(emb_weight, idx)
    return out5.transpose(2, 4, 0, 1, 3).reshape(BATCH, SEQ, DIM)


# speedup vs baseline: 1.0001x; 1.0001x over previous
import functools

import jax
import jax.numpy as jnp
from jax import lax
from jax.experimental import pallas as pl
from jax.experimental.pallas import tpu as pltpu
from jax.experimental.pallas import tpu_sc as plsc

NUM_TOKENS = 1000000
DIM = 64
BATCH = 4096
SEQ = 200

NC = 2
NS = 16
NW = NC * NS

TOTAL = BATCH * SEQ
IPG = 128
NCHUNK = TOTAL // IPG
CPW = NCHUNK // NW
BBLK = BATCH // IPG

_mesh = plsc.VectorSubcoreMesh(core_axis_name="c", subcore_axis_name="s")


@functools.partial(
    pl.kernel,
    out_type=jax.ShapeDtypeStruct((SEQ, DIM // 8, BBLK, 8 * IPG), jnp.float32),
    mesh=_mesh,
    scratch_types=[
        pltpu.VMEM((CPW, IPG), jnp.int32),
        pltpu.VMEM((IPG, DIM), jnp.float32),
        pltpu.VMEM((IPG, DIM), jnp.float32),
        pltpu.VMEM((DIM // 8, 8 * IPG), jnp.float32),
        pltpu.VMEM((DIM // 8, 8 * IPG), jnp.float32),
        pltpu.SemaphoreType.DMA,
        pltpu.SemaphoreType.DMA,
        pltpu.SemaphoreType.DMA,
        pltpu.SemaphoreType.DMA,
    ],
    compiler_params=pltpu.CompilerParams(
        use_tc_tiling_on_sc=False, needs_layout_passes=False
    ),
)
def _emb_lookup(table_hbm, idx_hbm, out_hbm, idx_v, buf0, buf1, bt0, bt1,
                gsem0, gsem1, osem0, osem1):
    wid = lax.axis_index("s") * NC + lax.axis_index("c")
    c_base = wid * CPW
    pltpu.sync_copy(idx_hbm.at[pl.ds(c_base, CPW)], idx_v)

    iota = lax.iota(jnp.int32, 16)
    dcols = [jnp.full((16,), d, jnp.int32) for d in range(DIM)]

    def fire_gather(g, buf, gsem):
        pltpu.async_copy(table_hbm.at[idx_v.at[g]], buf, gsem)

    def wait_gather(buf, gsem):
        pltpu.make_async_copy(table_hbm.at[idx_v.at[0]], buf, gsem).wait()

    def transpose(buf, bt):
        def bgloop(bg, carry):
            idx_b = iota + bg * 16
            boff = bg * 16
            for d in range(DIM):
                bt[d // 8, pl.ds((d % 8) * IPG + boff, 16)] = plsc.load_gather(
                    buf, [idx_b, dcols[d]]
                )
            return carry

        lax.fori_loop(0, IPG // 16, bgloop, 0)

    def fire_out(g, bt, osem):
        c = c_base + g
        s = c // BBLK
        bc = lax.rem(c, BBLK)
        pltpu.async_copy(bt, out_hbm.at[s, :, bc], osem)

    def wait_out(bt, osem):
        pltpu.make_async_copy(bt, out_hbm.at[0, :, 0], osem).wait()

    fire_gather(0, buf0, gsem0)

    def pair_body(t, carry):
        g0 = 2 * t
        fire_gather(g0 + 1, buf1, gsem1)
        wait_gather(buf0, gsem0)

        @pl.when(t > 0)
        def _():
            wait_out(bt0, osem0)

        transpose(buf0, bt0)
        fire_out(g0, bt0, osem0)

        @pl.when(g0 + 2 < CPW)
        def _():
            fire_gather(g0 + 2, buf0, gsem0)

        wait_gather(buf1, gsem1)

        @pl.when(t > 0)
        def _():
            wait_out(bt1, osem1)

        transpose(buf1, bt1)
        fire_out(g0 + 1, bt1, osem1)
        return carry

    lax.fori_loop(0, CPW // 2, pair_body, 0)
    wait_out(bt0, osem0)
    wait_out(bt1, osem1)


def kernel(x, emb_weight):
    idx = x.T.reshape(NCHUNK, IPG)
    out4 = _emb_lookup(emb_weight, idx)
    out5 = out4.reshape(SEQ, DIM // 8, BBLK, 8, IPG)
    return out5.transpose(2, 4, 0, 1, 3).reshape(BATCH, SEQ, DIM)


# speedup vs baseline: 1.4494x; 1.4493x over previous
import functools

import jax
import jax.numpy as jnp
from jax import lax
from jax.experimental import pallas as pl
from jax.experimental.pallas import tpu as pltpu
from jax.experimental.pallas import tpu_sc as plsc

NUM_TOKENS = 1000000
DIM = 64
BATCH = 4096
SEQ = 200

NC = 2
NS = 16
NW = NC * NS

TOTAL = BATCH * SEQ
IPG = 128
NIROW = TOTAL // IPG
IR_PW = NIROW // NW
GPC = 2
CHUNK = GPC * IPG
NCH = IR_PW // GPC
BBLK = BATCH // IPG

_mesh = plsc.VectorSubcoreMesh(core_axis_name="c", subcore_axis_name="s")


@functools.partial(
    pl.kernel,
    out_type=jax.ShapeDtypeStruct((SEQ, DIM // 8, BBLK, 8 * IPG), jnp.float32),
    mesh=_mesh,
    scratch_types=[
        pltpu.VMEM((IR_PW, IPG), jnp.int32),
        pltpu.VMEM((CHUNK, DIM), jnp.float32),
        pltpu.VMEM((CHUNK, DIM), jnp.float32),
        pltpu.VMEM((DIM // 8, GPC, 8 * IPG), jnp.float32),
        pltpu.VMEM((DIM // 8, GPC, 8 * IPG), jnp.float32),
        pltpu.SemaphoreType.DMA,
        pltpu.SemaphoreType.DMA,
        pltpu.SemaphoreType.DMA,
        pltpu.SemaphoreType.DMA,
    ],
    compiler_params=pltpu.CompilerParams(
        use_tc_tiling_on_sc=False, needs_layout_passes=False
    ),
)
def _emb_lookup(table_hbm, idx_hbm, out_hbm, idx_v, buf0, buf1, bt0, bt1,
                gsem0, gsem1, osem0, osem1):
    wid = lax.axis_index("s") * NC + lax.axis_index("c")
    c_base = wid * IR_PW
    pltpu.sync_copy(idx_hbm.at[pl.ds(c_base, IR_PW)], idx_v)

    iota = lax.iota(jnp.int32, 16)
    dcols = [jnp.full((16,), d, jnp.int32) for d in range(DIM)]

    def fire_gathers(u, buf, gsem):
        for j in range(GPC):
            pltpu.async_copy(
                table_hbm.at[idx_v.at[GPC * u + j]],
                buf.at[pl.ds(j * IPG, IPG)],
                gsem,
            )

    def wait_gathers(buf, gsem):
        for j in range(GPC):
            pltpu.make_async_copy(
                table_hbm.at[idx_v.at[j]],
                buf.at[pl.ds(j * IPG, IPG)],
                gsem,
            ).wait()

    def transpose(buf, bt):
        @plsc.parallel_loop(0, CHUNK // 16)
        def _(i):
            idx_b = iota + i * 16
            j = i // 8
            boff = (i % 8) * 16
            for dg in range(8):
                vs = [
                    plsc.load_gather(buf, [idx_b, dcols[dg * 8 + k]])
                    for k in range(8)
                ]
                for k in range(8):
                    bt[dg, j, pl.ds(k * IPG + boff, 16)] = vs[k]

    def fire_out(u, bt, osem):
        c = c_base + GPC * u
        s = c // BBLK
        bc = lax.rem(c, BBLK)
        pltpu.async_copy(bt, out_hbm.at[s, :, pl.ds(bc, GPC)], osem)

    def wait_out(bt, osem):
        pltpu.make_async_copy(bt, out_hbm.at[0, :, pl.ds(0, GPC)], osem).wait()

    fire_gathers(0, buf0, gsem0)

    def pair_body(t, carry):
        u0 = 2 * t
        fire_gathers(u0 + 1, buf1, gsem1)
        wait_gathers(buf0, gsem0)

        @pl.when(t > 0)
        def _():
            wait_out(bt0, osem0)

        transpose(buf0, bt0)
        fire_out(u0, bt0, osem0)

        @pl.when(u0 + 2 < NCH)
        def _():
            fire_gathers(u0 + 2, buf0, gsem0)

        wait_gathers(buf1, gsem1)

        @pl.when(t > 0)
        def _():
            wait_out(bt1, osem1)

        transpose(buf1, bt1)
        fire_out(u0 + 1, bt1, osem1)
        return carry

    lax.fori_loop(0, NCH // 2, pair_body, 0)
    wait_out(bt0, osem0)
    wait_out(bt1, osem1)


def kernel(x, emb_weight):
    idx = x.T.reshape(NIROW, IPG)
    out4 = _emb_lookup(emb_weight, idx)
    out5 = out4.reshape(SEQ, DIM // 8, BBLK, 8, IPG)
    return out5.transpose(2, 4, 0, 1, 3).reshape(BATCH, SEQ, DIM)


# speedup vs baseline: 2.3793x; 1.6415x over previous
import functools

import jax
import jax.numpy as jnp
from jax import lax
from jax.experimental import pallas as pl
from jax.experimental.pallas import tpu as pltpu
from jax.experimental.pallas import tpu_sc as plsc

NUM_TOKENS = 1000000
DIM = 64
BATCH = 4096
SEQ = 200

NC = 2
NS = 16
NW = NC * NS

TOTAL = BATCH * SEQ
IPG = 128
NIROW = TOTAL // IPG
IR_PW = NIROW // NW
GPC = 2
CHUNK = GPC * IPG
NCH = IR_PW // GPC
BBLK = BATCH // IPG

_mesh = plsc.VectorSubcoreMesh(core_axis_name="c", subcore_axis_name="s")


@functools.partial(
    pl.kernel,
    out_type=jax.ShapeDtypeStruct((SEQ, DIM // 8, BBLK, 8, IPG), jnp.float32),
    mesh=_mesh,
    scratch_types=[
        pltpu.VMEM((IR_PW, IPG), jnp.int32),
        pltpu.VMEM((CHUNK, DIM), jnp.float32),
        pltpu.VMEM((CHUNK, DIM), jnp.float32),
        pltpu.VMEM((DIM // 8, GPC, 8, IPG + 1), jnp.float32),
        pltpu.VMEM((DIM // 8, GPC, 8, IPG + 1), jnp.float32),
        pltpu.SemaphoreType.DMA,
        pltpu.SemaphoreType.DMA,
        pltpu.SemaphoreType.DMA,
        pltpu.SemaphoreType.DMA,
    ],
    compiler_params=pltpu.CompilerParams(
        use_tc_tiling_on_sc=False, needs_layout_passes=False
    ),
)
def _emb_lookup(table_hbm, idx_hbm, out_hbm, idx_v, buf0, buf1, bt0, bt1,
                gsem0, gsem1, osem0, osem1):
    wid = lax.axis_index("s") * NC + lax.axis_index("c")
    c_base = wid * IR_PW
    pltpu.sync_copy(idx_hbm.at[pl.ds(c_base, IR_PW)], idx_v)

    iota = lax.iota(jnp.int32, 16)
    dcols = [jnp.full((16,), d, jnp.int32) for d in range(DIM)]

    def fire_gathers(u, buf, gsem):
        for j in range(GPC):
            pltpu.async_copy(
                table_hbm.at[idx_v.at[GPC * u + j]],
                buf.at[pl.ds(j * IPG, IPG)],
                gsem,
            )

    def wait_gathers(buf, gsem):
        for j in range(GPC):
            pltpu.make_async_copy(
                table_hbm.at[idx_v.at[j]],
                buf.at[pl.ds(j * IPG, IPG)],
                gsem,
            ).wait()

    dt_idx = [lax.iota(jnp.int32, 16) // 8 + 2 * dg for dg in range(4)]
    d8_idx = lax.rem(lax.iota(jnp.int32, 16), 8)

    def transpose(buf, bt):
        @plsc.parallel_loop(0, CHUNK)
        def _(b):
            j = jnp.full((16,), b // IPG, jnp.int32)
            bcol = jnp.full((16,), lax.rem(b, IPG), jnp.int32)
            for dg in range(4):
                v = buf[b, pl.ds(16 * dg, 16)]
                plsc.store_scatter(bt, [dt_idx[dg], j, d8_idx, bcol], v)

    def fire_out(u, bt, osem):
        c = c_base + GPC * u
        s = c // BBLK
        bc = lax.rem(c, BBLK)
        pltpu.async_copy(bt.at[:, :, :, pl.ds(0, IPG)],
                         out_hbm.at[s, :, pl.ds(bc, GPC)], osem)

    def wait_out(bt, osem):
        pltpu.make_async_copy(bt.at[:, :, :, pl.ds(0, IPG)],
                              out_hbm.at[0, :, pl.ds(0, GPC)], osem).wait()

    fire_gathers(0, buf0, gsem0)

    def pair_body(t, carry):
        u0 = 2 * t
        fire_gathers(u0 + 1, buf1, gsem1)
        wait_gathers(buf0, gsem0)

        @pl.when(t > 0)
        def _():
            wait_out(bt0, osem0)

        transpose(buf0, bt0)
        fire_out(u0, bt0, osem0)

        @pl.when(u0 + 2 < NCH)
        def _():
            fire_gathers(u0 + 2, buf0, gsem0)

        wait_gathers(buf1, gsem1)

        @pl.when(t > 0)
        def _():
            wait_out(bt1, osem1)

        transpose(buf1, bt1)
        fire_out(u0 + 1, bt1, osem1)
        return carry

    lax.fori_loop(0, NCH // 2, pair_body, 0)
    wait_out(bt0, osem0)
    wait_out(bt1, osem1)


def kernel(x, emb_weight):
    idx = x.T.reshape(NIROW, IPG)
    out5 = _emb_lookup(emb_weight, idx)
    return out5.transpose(2, 4, 0, 1, 3).reshape(BATCH, SEQ, DIM)


# speedup vs baseline: 2.4129x; 1.0142x over previous
import functools

import jax
import jax.numpy as jnp
from jax import lax
from jax.experimental import pallas as pl
from jax.experimental.pallas import tpu as pltpu
from jax.experimental.pallas import tpu_sc as plsc

NUM_TOKENS = 1000000
DIM = 64
BATCH = 4096
SEQ = 200

NC = 2
NS = 16
NW = NC * NS

TOTAL = BATCH * SEQ
IPG = 128
NIROW = TOTAL // IPG
IR_PW = NIROW // NW
BBLK = BATCH // IPG
DEPTH = 4

_mesh = plsc.VectorSubcoreMesh(core_axis_name="c", subcore_axis_name="s")


@functools.partial(
    pl.kernel,
    out_type=jax.ShapeDtypeStruct((SEQ, DIM // 8, BBLK, 8, IPG), jnp.float32),
    mesh=_mesh,
    scratch_types=[
        pltpu.VMEM((IR_PW, IPG), jnp.int32),
        [pltpu.VMEM((IPG, DIM), jnp.float32) for _ in range(DEPTH)],
        [pltpu.VMEM((DIM // 8, 1, 8, IPG + 1), jnp.float32)
         for _ in range(DEPTH)],
        [pltpu.SemaphoreType.DMA for _ in range(DEPTH)],
        [pltpu.SemaphoreType.DMA for _ in range(DEPTH)],
    ],
    compiler_params=pltpu.CompilerParams(
        use_tc_tiling_on_sc=False, needs_layout_passes=False
    ),
)
def _emb_lookup(table_hbm, idx_hbm, out_hbm, idx_v, bufs, bts, gsems, osems):
    wid = lax.axis_index("s") * NC + lax.axis_index("c")
    c_base = wid * IR_PW
    pltpu.sync_copy(idx_hbm.at[pl.ds(c_base, IR_PW)], idx_v)

    dt_idx = [lax.iota(jnp.int32, 16) // 8 + 2 * dg for dg in range(4)]
    d8_idx = lax.rem(lax.iota(jnp.int32, 16), 8)
    zero = jnp.zeros((16,), jnp.int32)

    def fire_gather(u, buf, gsem):
        pltpu.async_copy(table_hbm.at[idx_v.at[u]], buf, gsem)

    def wait_gather(buf, gsem):
        pltpu.make_async_copy(table_hbm.at[idx_v.at[0]], buf, gsem).wait()

    def transpose(buf, bt):
        @plsc.parallel_loop(0, IPG)
        def _(b):
            bcol = jnp.full((16,), b, jnp.int32)
            for dg in range(4):
                v = buf[b, pl.ds(16 * dg, 16)]
                plsc.store_scatter(bt, [dt_idx[dg], zero, d8_idx, bcol], v)

    def fire_out(u, bt, osem):
        c = c_base + u
        s = c // BBLK
        bc = lax.rem(c, BBLK)
        pltpu.async_copy(bt.at[:, :, :, pl.ds(0, IPG)],
                         out_hbm.at[s, :, pl.ds(bc, 1)], osem)

    def wait_out(bt, osem):
        pltpu.make_async_copy(bt.at[:, :, :, pl.ds(0, IPG)],
                              out_hbm.at[0, :, pl.ds(0, 1)], osem).wait()

    for r in range(DEPTH):
        fire_gather(r, bufs[r], gsems[r])

    def body(t, carry):
        for r in range(DEPTH):
            u = DEPTH * t + r
            wait_gather(bufs[r], gsems[r])

            @pl.when(t > 0)
            def _():
                wait_out(bts[r], osems[r])

            transpose(bufs[r], bts[r])
            fire_out(u, bts[r], osems[r])

            @pl.when(u + DEPTH < IR_PW)
            def _():
                fire_gather(u + DEPTH, bufs[r], gsems[r])

        return carry

    lax.fori_loop(0, IR_PW // DEPTH, body, 0)
    for r in range(DEPTH):
        wait_out(bts[r], osems[r])


def kernel(x, emb_weight):
    idx = x.T.reshape(NIROW, IPG)
    out5 = _emb_lookup(emb_weight, idx)
    return out5.transpose(2, 4, 0, 1, 3).reshape(BATCH, SEQ, DIM)
